# baseline (speedup 1.0000x reference)
import functools
import math

import jax
import jax.numpy as jnp
from jax import lax
from jax.experimental import pallas as pl
from jax.experimental.pallas import tpu as pltpu
from jax.experimental.pallas import tpu_sc as plsc

N = 84
NH = 256
DT = 0.1
NSTEP = 80
NP = 96
NCH = NP // 16
T = NH + NSTEP + 1

E0, KAPPA, GAMMA, TAU_O, ALPHA = 0.4, 0.65, 0.41, 0.98, 0.32
V0, K1, K2, K3 = 0.02, 2.8, 0.8, 0.48
BOLD_DT = DT / 10000.0
P1 = 1.0 / ALPHA
LOG_1ME0 = math.log(1.0 - E0)
LN2 = 0.6931471805599453
SQRT2 = 1.4142135623730951


def _vlog(v):
    bits = plsc.bitcast(v, jnp.int32)
    e = jnp.right_shift(bits, 23) & 0xFF
    m = plsc.bitcast((bits & 0x7FFFFF) | 0x3F800000, jnp.float32)
    big = m > SQRT2
    m = jnp.where(big, m * 0.5, m)
    ef = (e - 127).astype(jnp.float32) + jnp.where(big, 1.0, 0.0)
    t = (m - 1.0) / (m + 1.0)
    t2 = t * t
    lnm = 2.0 * t * (1.0 + t2 * (1.0 / 3.0 + t2 * (1.0 / 5.0 + t2 * (1.0 / 7.0))))
    return ef * LN2 + lnm


_mesh = plsc.VectorSubcoreMesh(core_axis_name="c", subcore_axis_name="s")


@functools.partial(
    pl.kernel,
    out_type=jax.ShapeDtypeStruct((2 * NP,), jnp.float32),
    mesh=_mesh,
    compiler_params=pltpu.CompilerParams(needs_layout_passes=False),
    scratch_types=[
        pltpu.VMEM((T * NP,), jnp.float32),
        pltpu.VMEM((N * 16,), jnp.float32),
        pltpu.VMEM((N * 16,), jnp.int32),
        pltpu.VMEM((NSTEP * NP,), jnp.float32),
        pltpu.VMEM((NSTEP * NP,), jnp.float32),
        pltpu.VMEM((5 * NP,), jnp.float32),
        pltpu.VMEM((2 * 16,), jnp.float32),
        pltpu.VMEM((16,), jnp.float32),
        pltpu.VMEM((2 * 16,), jnp.float32),
        pltpu.VMEM_SHARED((2 * NP,), jnp.float32),
        pltpu.SemaphoreType.DMA,
    ],
)
def _sc_integrate(w_hbm, base_hbm, dwx_hbm, dwy_hbm, misc_hbm, hist0_hbm,
                  out_hbm, hist_v, wb_v, bb_v, dwx_v, dwy_v, misc_v, st_v,
                  pub_v, bold_v, stage, csem):
    cid = lax.axis_index("c")
    sid = lax.axis_index("s")
    work = jnp.logical_and(cid == 0, sid < NCH)

    @pl.when(work)
    def _setup():
        pltpu.sync_copy(w_hbm.at[pl.ds(sid * N * 16, N * 16)], wb_v)
        pltpu.sync_copy(base_hbm.at[pl.ds(sid * N * 16, N * 16)], bb_v)
        pltpu.sync_copy(dwx_hbm, dwx_v)
        pltpu.sync_copy(dwy_hbm, dwy_v)
        pltpu.sync_copy(misc_hbm, misc_v)
        pltpu.sync_copy(hist0_hbm, hist_v)
        st_v[pl.ds(0, 16)] = misc_v[pl.ds(2 * NP + sid * 16, 16)]
        st_v[pl.ds(16, 16)] = misc_v[pl.ds(3 * NP + sid * 16, 16)]

    def step(gs, carry):
        slot = (NH + 1 + gs) * NP
        bsel = lax.rem(gs, 2) * NP

        @pl.when(work)
        def _compute():
            for ci in range(NCH):
                hist_v[pl.ds(slot + ci * 16, 16)] = \
                    dwx_v[pl.ds(gs * NP + ci * 16, 16)]

            def rows(off4, wv_off, a1, a2, count4):
                def gbody(k, acc):
                    b1, b2 = acc
                    for u in range(4):
                        off = off4 + k * (4 * 16) + u * 16
                        wv = wb_v[pl.ds(off, 16)]
                        iv = bb_v[pl.ds(off, 16)] + gs * NP
                        v1 = plsc.load_gather(hist_v, [iv])
                        v2 = plsc.load_gather(hist_v, [iv + NP])
                        b1 = b1 + wv * v1
                        b2 = b2 + wv * v2
                    return b1, b2
                return lax.fori_loop(0, count4, gbody, (a1, a2))

            nsafe4 = misc_v[pl.ds(4 * NP + sid * 16, 16)].astype(jnp.int32)[0]
            z = jnp.zeros((16,), jnp.float32)
            c1, c2 = rows(0, 0, z, z, nsafe4)

            @pl.when(gs > 0)
            def _wait_prev():
                pbsel = NP - bsel
                pltpu.make_async_copy(
                    stage.at[pl.ds(pbsel, NP)],
                    hist_v.at[pl.ds(slot - NP, NP)], csem).wait()

            def tail(k, acc):
                b1, b2 = acc
                off = k * 16
                wv = wb_v[pl.ds(off, 16)]
                iv = bb_v[pl.ds(off, 16)] + gs * NP
                v1 = plsc.load_gather(hist_v, [iv])
                v2 = plsc.load_gather(hist_v, [iv + NP])
                return b1 + wv * v1, b2 + wv * v2
            c1, c2 = lax.fori_loop(nsafe4 * 4, N, tail, (c1, c2))

            X = st_v[pl.ds(0, 16)]
            Y = st_v[pl.ds(16, 16)]
            av = misc_v[pl.ds(sid * 16, 16)]
            tv = misc_v[pl.ds(NP + sid * 16, 16)]
            dwx = dwx_v[pl.ds(gs * NP + sid * 16, 16)]
            dwy = dwy_v[pl.ds(gs * NP + sid * 16, 16)]
            dX1 = tv * (X - X * X * X / 3.0 + Y) + c1
            dY1 = (av - X) / tv
            xiX = X + DT * dX1 + dwx
            xiY = Y + DT * dY1 + dwy
            dX2 = tv * (xiX - xiX * xiX * xiX / 3.0 + xiY) + c2
            dY2 = (av - xiX) / tv
            nX = X + (DT * 0.5) * (dX1 + dX2) + dwx
            nY = Y + (DT * 0.5) * (dY1 + dY2) + dwy
            st_v[pl.ds(0, 16)] = nX
            st_v[pl.ds(16, 16)] = nY
            pub_v[pl.ds(0, 16)] = nX
            pltpu.sync_copy(pub_v, stage.at[pl.ds(bsel + sid * 16, 16)])

        plsc.subcore_barrier()

        @pl.when(work)
        def _collect():
            pltpu.async_copy(stage.at[pl.ds(bsel, NP)],
                             hist_v.at[pl.ds(slot, NP)], csem)
        return carry

    @pl.when(cid == 0)
    def _loop():
        lax.fori_loop(0, NSTEP, step, 0)

    @pl.when(work)
    def _bold():
        lastb = lax.rem(NSTEP - 1, 2) * NP
        pltpu.make_async_copy(
            stage.at[pl.ds(lastb, NP)],
            hist_v.at[pl.ds((NH + NSTEP) * NP, NP)], csem).wait()

        def dfun(st, x):
            s, f, v, q = st
            lv = _vlog(v)
            vp1 = jnp.exp(P1 * lv)
            vp2 = jnp.exp((P1 - 1.0) * lv)
            ds = x - KAPPA * s - GAMMA * (f - 1.0)
            dv = (f - vp1) / TAU_O
            dq = (f * (1.0 - jnp.exp(LOG_1ME0 / f)) / E0 - q * vp2) / TAU_O
            return ds, s, dv, dq

        def heun_b(i, st):
            x = hist_v[pl.ds((NH + 1) * NP + i * NP + sid * 16, 16)]
            d1 = dfun(st, x)
            xi = tuple(a + BOLD_DT * b for a, b in zip(st, d1))
            d2 = dfun(xi, x)
            return tuple(a + BOLD_DT * 0.5 * (b + c)
                         for a, b, c in zip(st, d1, d2))

        def bold_of(st):
            _, _, v, q = st
            return V0 * (K1 * (1.0 - q) + K2 * (1.0 - q / v) + K3 * (1.0 - v))

        one = jnp.ones((16,), jnp.float32)
        st = lax.fori_loop(0, 40, heun_b, (one, one, one, one))
        bold_v[pl.ds(0, 16)] = bold_of(st)
        st = lax.fori_loop(40, 80, heun_b, st)
        bold_v[pl.ds(16, 16)] = bold_of(st)
        pltpu.sync_copy(bold_v.at[pl.ds(0, 16)],
                        out_hbm.at[pl.ds(sid * 16, 16)])
        pltpu.sync_copy(bold_v.at[pl.ds(16, 16)],
                        out_hbm.at[pl.ds(NP + sid * 16, 16)])


def kernel(region_pars, g, Wt, lags, ix_lag_from, init_state, noise):
    f32 = jnp.float32
    W = Wt[:, :, 0] * g[0]
    base = (NH - lags) * NP + ix_lag_from
    unsafe = (lags <= 1).astype(jnp.int32)
    c_safe = jnp.cumsum(1 - unsafe, axis=1)
    c_uns = jnp.cumsum(unsafe, axis=1)
    n_safe = c_safe[:, -1:]
    pos = jnp.where(unsafe == 0, c_safe - 1, n_safe + c_uns - 1)
    onehot = (pos[:, None, :] == jnp.arange(N)[None, :, None]).astype(f32)
    W = jnp.einsum("dkj,dj->dk", onehot, W)
    base = jnp.einsum("dkj,dj->dk", onehot, base.astype(f32)).astype(jnp.int32)
    ns_pad = jnp.full((NP,), N, jnp.int32).at[:N].set(n_safe[:, 0])
    ns4 = jnp.min(ns_pad.reshape(NCH, 16), axis=1) // 4

    W96 = jnp.zeros((NP, N), f32).at[:N].set(W)
    base96 = jnp.zeros((NP, N), jnp.int32).at[:N].set(base)
    w_rows = W96.reshape(NCH, 16, N).transpose(0, 2, 1).reshape(-1)
    b_rows = base96.reshape(NCH, 16, N).transpose(0, 2, 1).reshape(-1)
    dW = noise.reshape(NSTEP, N, 2) * f32(math.sqrt(DT) * 0.01)
    dwx = jnp.zeros((NSTEP, NP), f32).at[:, :N].set(dW[:, :, 0]).reshape(-1)
    dwy = jnp.zeros((NSTEP, NP), f32).at[:, :N].set(dW[:, :, 1]).reshape(-1)
    misc = jnp.zeros((5, NP), f32)
    misc = misc.at[0, :N].set(region_pars[:, 0])
    misc = misc.at[1].set(1.0)
    misc = misc.at[1, :N].set(region_pars[:, 1] + 1.0)
    misc = misc.at[2, :N].set(init_state[:, 0])
    misc = misc.at[3, :N].set(init_state[:, 1])
    misc = misc.at[4].set(
        jnp.repeat(ns4.astype(f32), 16, total_repeat_length=NP))
    hist0 = jnp.zeros((T, NP), f32)
    hist0 = hist0.at[:NH + 1, :N].set(
        jnp.broadcast_to(init_state[:, 0], (NH + 1, N)))

    bold = _sc_integrate(w_rows, b_rows, dwx, dwy,
                         misc.reshape(-1), hist0.reshape(-1))
    return bold.reshape(2, NP)[:, :N].reshape(2, N, 1)


# speedup vs baseline: 2.3889x; 2.3889x over previous
import functools
import math

import jax
import jax.numpy as jnp
from jax import lax
from jax.experimental import pallas as pl
from jax.experimental.pallas import tpu as pltpu
from jax.experimental.pallas import tpu_sc as plsc

N = 84
NH = 256
DT = 0.1
NSTEP = 80
NP = 96
NCH = NP // 16
T = NH + NSTEP + 1

E0, KAPPA, GAMMA, TAU_O, ALPHA = 0.4, 0.65, 0.41, 0.98, 0.32
V0, K1, K2, K3 = 0.02, 2.8, 0.8, 0.48
BOLD_DT = DT / 10000.0
P1 = 1.0 / ALPHA
LOG_1ME0 = math.log(1.0 - E0)
LN2 = 0.6931471805599453
SQRT2 = 1.4142135623730951

_mesh = plsc.VectorSubcoreMesh(core_axis_name="c", subcore_axis_name="s")


def _vlog(v):
    bits = plsc.bitcast(v, jnp.int32)
    e = jnp.right_shift(bits, 23) & 0xFF
    m = plsc.bitcast((bits & 0x7FFFFF) | 0x3F800000, jnp.float32)
    big = m > SQRT2
    m = jnp.where(big, m * 0.5, m)
    ef = (e - 127).astype(jnp.float32) + jnp.where(big, 1.0, 0.0)
    t = (m - 1.0) / (m + 1.0)
    t2 = t * t
    lnm = 2.0 * t * (1.0 + t2 * (1.0 / 3.0 + t2 * (1.0 / 5.0 + t2 * (1.0 / 7.0))))
    return ef * LN2 + lnm


@functools.partial(
    pl.kernel,
    out_type=jax.ShapeDtypeStruct((2 * NP,), jnp.float32),
    mesh=_mesh,
    compiler_params=pltpu.CompilerParams(needs_layout_passes=False),
    scratch_types=[
        pltpu.VMEM((T * NP,), jnp.float32),
        pltpu.VMEM((N * 16,), jnp.float32),
        pltpu.VMEM((N * 16,), jnp.int32),
        pltpu.VMEM((NSTEP * NP,), jnp.float32),
        pltpu.VMEM((NSTEP * NP,), jnp.float32),
        pltpu.VMEM((4 * NP,), jnp.float32),
        pltpu.VMEM((2 * 16,), jnp.float32),
        pltpu.VMEM((16,), jnp.float32),
        pltpu.VMEM((2 * 16,), jnp.float32),
        pltpu.VMEM_SHARED((2 * NP,), jnp.float32),
    ],
)
def _sc_integrate(w_hbm, base_hbm, dwx_hbm, dwy_hbm, misc_hbm, hist0_hbm,
                  out_hbm, hist_v, wb_v, bb_v, dwx_v, dwy_v, misc_v, st_v,
                  pub_v, bold_v, stage):
    cid = lax.axis_index("c")
    sid = lax.axis_index("s")
    work = jnp.logical_and(cid == 0, sid < NCH)

    @pl.when(work)
    def _setup():
        pltpu.sync_copy(w_hbm.at[pl.ds(sid * N * 16, N * 16)], wb_v)
        pltpu.sync_copy(base_hbm.at[pl.ds(sid * N * 16, N * 16)], bb_v)
        pltpu.sync_copy(dwx_hbm, dwx_v)
        pltpu.sync_copy(dwy_hbm, dwy_v)
        pltpu.sync_copy(misc_hbm, misc_v)
        pltpu.sync_copy(hist0_hbm, hist_v)
        st_v[pl.ds(0, 16)] = misc_v[pl.ds(2 * NP + sid * 16, 16)]
        st_v[pl.ds(16, 16)] = misc_v[pl.ds(3 * NP + sid * 16, 16)]

    def step(gs, carry):
        slot = (NH + 1 + gs) * NP
        bsel = lax.rem(gs, 2) * NP

        @pl.when(work)
        def _compute():
            for ci in range(NCH):
                hist_v[pl.ds(slot + ci * 16, 16)] = \
                    dwx_v[pl.ds(gs * NP + ci * 16, 16)]

            def gbody(k, acc):
                a1, a2 = acc
                for u in range(6):
                    off = k * (6 * 16) + u * 16
                    wv = wb_v[pl.ds(off, 16)]
                    iv = bb_v[pl.ds(off, 16)] + gs * NP
                    v1 = plsc.load_gather(hist_v, [iv])
                    v2 = plsc.load_gather(hist_v, [iv + NP])
                    a1 = a1 + wv * v1
                    a2 = a2 + wv * v2
                return a1, a2
            z = jnp.zeros((16,), jnp.float32)
            c1, c2 = lax.fori_loop(0, N // 6, gbody, (z, z))

            X = st_v[pl.ds(0, 16)]
            Y = st_v[pl.ds(16, 16)]
            av = misc_v[pl.ds(sid * 16, 16)]
            tv = misc_v[pl.ds(NP + sid * 16, 16)]
            dwx = dwx_v[pl.ds(gs * NP + sid * 16, 16)]
            dwy = dwy_v[pl.ds(gs * NP + sid * 16, 16)]
            dX1 = tv * (X - X * X * X / 3.0 + Y) + c1
            dY1 = (av - X) / tv
            xiX = X + DT * dX1 + dwx
            xiY = Y + DT * dY1 + dwy
            dX2 = tv * (xiX - xiX * xiX * xiX / 3.0 + xiY) + c2
            dY2 = (av - xiX) / tv
            nX = X + (DT * 0.5) * (dX1 + dX2) + dwx
            nY = Y + (DT * 0.5) * (dY1 + dY2) + dwy
            st_v[pl.ds(0, 16)] = nX
            st_v[pl.ds(16, 16)] = nY
            pub_v[pl.ds(0, 16)] = nX
            pltpu.sync_copy(pub_v, stage.at[pl.ds(bsel + sid * 16, 16)])

        plsc.subcore_barrier()

        @pl.when(work)
        def _collect():
            pltpu.sync_copy(stage.at[pl.ds(bsel, NP)],
                            hist_v.at[pl.ds(slot, NP)])
        return carry

    @pl.when(cid == 0)
    def _loop():
        lax.fori_loop(0, NSTEP, step, 0)

    @pl.when(work)
    def _bold():
        def dfun(st, x):
            s, f, v, q = st
            lv = _vlog(v)
            vp1 = jnp.exp(P1 * lv)
            vp2 = vp1 / v
            ds = x - KAPPA * s - GAMMA * (f - 1.0)
            dv = (f - vp1) / TAU_O
            dq = (f * (1.0 - jnp.exp(LOG_1ME0 / f)) / E0 - q * vp2) / TAU_O
            return ds, s, dv, dq

        def heun_b(i, st):
            x = hist_v[pl.ds((NH + 1) * NP + i * NP + sid * 16, 16)]
            d1 = dfun(st, x)
            xi = tuple(a + BOLD_DT * b for a, b in zip(st, d1))
            d2 = dfun(xi, x)
            return tuple(a + BOLD_DT * 0.5 * (b + c)
                         for a, b, c in zip(st, d1, d2))

        def bold_of(st):
            _, _, v, q = st
            return V0 * (K1 * (1.0 - q) + K2 * (1.0 - q / v) + K3 * (1.0 - v))

        one = jnp.ones((16,), jnp.float32)
        st = lax.fori_loop(0, 40, heun_b, (one, one, one, one))
        bold_v[pl.ds(0, 16)] = bold_of(st)
        st = lax.fori_loop(40, 80, heun_b, st)
        bold_v[pl.ds(16, 16)] = bold_of(st)
        pltpu.sync_copy(bold_v.at[pl.ds(0, 16)],
                        out_hbm.at[pl.ds(sid * 16, 16)])
        pltpu.sync_copy(bold_v.at[pl.ds(16, 16)],
                        out_hbm.at[pl.ds(NP + sid * 16, 16)])


def kernel(region_pars, g, Wt, lags, ix_lag_from, init_state, noise):
    f32 = jnp.float32
    W = Wt[:, :, 0] * g[0]
    W96 = jnp.zeros((NP, N), f32).at[:N].set(W)
    base = (NH - lags) * NP + ix_lag_from
    base96 = jnp.zeros((NP, N), jnp.int32).at[:N].set(base)
    w_rows = W96.reshape(NCH, 16, N).transpose(0, 2, 1).reshape(-1)
    b_rows = base96.reshape(NCH, 16, N).transpose(0, 2, 1).reshape(-1)
    dW = noise.reshape(NSTEP, N, 2) * f32(math.sqrt(DT) * 0.01)
    dwx = jnp.zeros((NSTEP, NP), f32).at[:, :N].set(dW[:, :, 0]).reshape(-1)
    dwy = jnp.zeros((NSTEP, NP), f32).at[:, :N].set(dW[:, :, 1]).reshape(-1)
    misc = jnp.zeros((4, NP), f32)
    misc = misc.at[0, :N].set(region_pars[:, 0])
    misc = misc.at[1].set(1.0)
    misc = misc.at[1, :N].set(region_pars[:, 1] + 1.0)
    misc = misc.at[2, :N].set(init_state[:, 0])
    misc = misc.at[3, :N].set(init_state[:, 1])
    hist0 = jnp.zeros((T, NP), f32)
    hist0 = hist0.at[:NH + 1, :N].set(
        jnp.broadcast_to(init_state[:, 0], (NH + 1, N)))

    bold = _sc_integrate(w_rows, b_rows, dwx, dwy,
                         misc.reshape(-1), hist0.reshape(-1))
    return bold.reshape(2, NP)[:, :N].reshape(2, N, 1)
